# Initial kernel scaffold; baseline (speedup 1.0000x reference)
import jax
import jax.numpy as jnp
from jax.experimental import pallas as pl


def kernel(x, edge_index, batch, t, W1, W2, W3, Wg, E1_w, E1_b, E2_w, E2_b):
    raise NotImplementedError("write your pallas kernel here")


# speedup vs baseline: 34.7601x; 34.7601x over previous
import functools

import jax
import jax.numpy as jnp
from jax import lax
from jax.experimental import pallas as pl
from jax.experimental.pallas import tpu as pltpu
from jax.experimental.pallas import tpu_sc as plsc

NC = 2
NS = 16
NW = NC * NS

CHUNK = 2048
LANES = 16


def _sc_mesh():
  return plsc.VectorSubcoreMesh(
      core_axis_name="c", subcore_axis_name="s", num_cores=NC, num_subcores=NS)


_SC_PARAMS = pltpu.CompilerParams(use_tc_tiling_on_sc=False,
                                 needs_layout_passes=False)


def _make_gather_kernel(n_pad, e_pad):
  t_per_tile = e_pad // NW
  n_chunks = t_per_tile // CHUNK

  @functools.partial(
      pl.kernel,
      out_type=[jax.ShapeDtypeStruct((e_pad,), jnp.float32)],
      mesh=_sc_mesh(),
      compiler_params=_SC_PARAMS,
      scratch_types=[
          pltpu.VMEM((n_pad,), jnp.float32),
          pltpu.VMEM((CHUNK,), jnp.int32),
          pltpu.VMEM((CHUNK,), jnp.int32),
          pltpu.VMEM((CHUNK,), jnp.float32),
      ],
  )
  def gather_kernel(table_hbm, src_hbm, dst_hbm, d_out,
                    table_v, src_v, dst_v, d_v):
    cid = lax.axis_index("c")
    sid = lax.axis_index("s")
    wid = sid * NC + cid

    pltpu.sync_copy(table_hbm, table_v)

    def chunk_body(g, carry):
      eb = wid * t_per_tile + g * CHUNK
      pltpu.sync_copy(src_hbm.at[pl.ds(eb, CHUNK)], src_v)
      pltpu.sync_copy(dst_hbm.at[pl.ds(eb, CHUNK)], dst_v)

      def vec_body(k, carry2):
        o = k * LANES
        a = plsc.load_gather(table_v, [src_v[pl.ds(o, LANES)]])
        b = plsc.load_gather(table_v, [dst_v[pl.ds(o, LANES)]])
        d_v[pl.ds(o, LANES)] = a - b
        return carry2

      lax.fori_loop(0, CHUNK // LANES, vec_body, 0)
      pltpu.sync_copy(d_v, d_out.at[pl.ds(eb, CHUNK)])
      return carry

    lax.fori_loop(0, n_chunks, chunk_body, 0)

  return gather_kernel


def _make_scatter_kernel(n_pad, e_pad):
  t_per_tile = e_pad // NW
  n_chunks = t_per_tile // CHUNK

  @functools.partial(
      pl.kernel,
      out_type=[jax.ShapeDtypeStruct((3, NW, n_pad), jnp.float32)],
      mesh=_sc_mesh(),
      compiler_params=_SC_PARAMS,
      scratch_types=[
          pltpu.VMEM((n_pad,), jnp.float32),
          pltpu.VMEM((CHUNK,), jnp.int32),
          pltpu.VMEM((CHUNK,), jnp.float32),
      ],
  )
  def scatter_kernel(m0_hbm, m1_hbm, dst_hbm, part_out, acc_v, dst_v, m_v):
    cid = lax.axis_index("c")
    sid = lax.axis_index("s")
    wid = sid * NC + cid
    zeros16 = jnp.zeros((LANES,), jnp.float32)
    ones16 = jnp.ones((LANES,), jnp.float32)

    for c, m_hbm in ((0, m0_hbm), (1, m1_hbm), (2, None)):
      def zero_body(k, carry):
        acc_v[pl.ds(k * LANES, LANES)] = zeros16
        return carry

      lax.fori_loop(0, n_pad // LANES, zero_body, 0)

      def chunk_body(g, carry, m_hbm=m_hbm):
        eb = wid * t_per_tile + g * CHUNK
        pltpu.sync_copy(dst_hbm.at[pl.ds(eb, CHUNK)], dst_v)
        if m_hbm is not None:
          pltpu.sync_copy(m_hbm.at[pl.ds(eb, CHUNK)], m_v)

        def vec_body(k, carry2, use_m=m_hbm is not None):
          o = k * LANES
          vals = m_v[pl.ds(o, LANES)] if use_m else ones16
          plsc.addupdate_scatter(acc_v, [dst_v[pl.ds(o, LANES)]], vals)
          return carry2

        lax.fori_loop(0, CHUNK // LANES, vec_body, 0)
        return carry

      lax.fori_loop(0, n_chunks, chunk_body, 0)
      pltpu.sync_copy(acc_v, part_out.at[c, wid])

  return scatter_kernel


def _mlp_block(d0_ref, d1_ref, w1t_ref, w2t_ref, w3t_ref, m0_ref, m1_ref):
  d0 = d0_ref[0]
  d1 = d1_ref[0]
  h = jnp.tanh(w1t_ref[:, 0:1] * d0 + w1t_ref[:, 1:2] * d1)
  h = jnp.tanh(jnp.dot(w2t_ref[...], h, preferred_element_type=jnp.float32))
  m = jnp.dot(w3t_ref[...], h, preferred_element_type=jnp.float32)
  m0_ref[0] = m[0:1, :]
  m1_ref[0] = m[1:2, :]


def _combine_block(p_ref, o_ref):
  cnt = jnp.sum(p_ref[2], axis=0, keepdims=True)
  denom = jnp.maximum(cnt, 1.0)
  o_ref[0:1, :] = jnp.sum(p_ref[0], axis=0, keepdims=True) / denom
  o_ref[1:2, :] = jnp.sum(p_ref[1], axis=0, keepdims=True) / denom


def kernel(x, edge_index, batch, t, W1, W2, W3, Wg, E1_w, E1_b, E2_w, E2_b):
  n = x.shape[0]
  e = edge_index.shape[1]

  slab = NW * CHUNK
  e_pad = ((e + slab - 1) // slab) * slab
  n_pad = ((n + 1 + CHUNK - 1) // CHUNK) * CHUNK

  pad = e_pad - e
  src = jnp.concatenate([edge_index[0], jnp.full((pad,), n, jnp.int32)])
  dst = jnp.concatenate([edge_index[1], jnp.full((pad,), n, jnp.int32)])
  xp = jnp.pad(x, ((0, n_pad - n), (0, 0)))
  x0 = xp[:, 0]
  x1 = xp[:, 1]

  gather = _make_gather_kernel(n_pad, e_pad)
  (d0,) = gather(x0, src, dst)
  (d1,) = gather(x1, src, dst)

  bt = 8192
  nb = e_pad // bt
  m0, m1 = pl.pallas_call(
      _mlp_block,
      grid=(nb,),
      in_specs=[
          pl.BlockSpec((1, 1, bt), lambda i: (i, 0, 0)),
          pl.BlockSpec((1, 1, bt), lambda i: (i, 0, 0)),
          pl.BlockSpec((64, 2), lambda i: (0, 0)),
          pl.BlockSpec((64, 64), lambda i: (0, 0)),
          pl.BlockSpec((2, 64), lambda i: (0, 0)),
      ],
      out_specs=[
          pl.BlockSpec((1, 1, bt), lambda i: (i, 0, 0)),
          pl.BlockSpec((1, 1, bt), lambda i: (i, 0, 0)),
      ],
      out_shape=[
          jax.ShapeDtypeStruct((nb, 1, bt), jnp.float32),
          jax.ShapeDtypeStruct((nb, 1, bt), jnp.float32),
      ],
  )(d0.reshape(nb, 1, bt), d1.reshape(nb, 1, bt), W1.T, W2.T, W3.T)

  (part,) = _make_scatter_kernel(n_pad, e_pad)(
      m0.reshape(e_pad), m1.reshape(e_pad), dst)

  bn = 2048
  outT = pl.pallas_call(
      _combine_block,
      grid=(n_pad // bn,),
      in_specs=[pl.BlockSpec((3, NW, bn), lambda i: (0, 0, i))],
      out_specs=pl.BlockSpec((2, bn), lambda i: (0, i)),
      out_shape=jax.ShapeDtypeStruct((2, n_pad), jnp.float32),
  )(part)

  return outT.T[:n]
